# baseline (speedup 1.0000x reference)
import functools

import jax
import jax.numpy as jnp
from jax import lax
from jax.experimental import pallas as pl
from jax.experimental.pallas import tpu as pltpu
from jax.experimental.pallas import tpu_sc as plsc

B = 16384
D = 32
H = 128
O = 5
BB = 2048
PK = 128 // D


@functools.cache
def _gather_fn(nu4, nm4):
    info = plsc.get_sparse_core_info()
    NC, NS = info.num_cores, info.num_subcores
    NW = NC * NS
    b_per_w = B // NW
    mesh = plsc.VectorSubcoreMesh(core_axis_name="c", subcore_axis_name="s")

    @functools.partial(
        pl.kernel,
        mesh=mesh,
        out_type=(
            jax.ShapeDtypeStruct((B, 128), jnp.float32),
            jax.ShapeDtypeStruct((B, 128), jnp.float32),
        ),
        scratch_types=[
            pltpu.VMEM((b_per_w,), jnp.int32),
            pltpu.VMEM((b_per_w,), jnp.int32),
            pltpu.VMEM((b_per_w, 128), jnp.float32),
            pltpu.SemaphoreType.DMA,
        ],
        compiler_params=pltpu.CompilerParams(use_tc_tiling_on_sc=True),
    )
    def gather_k(utab, mtab, uid, mid, gu_out, gm_out,
                 uidx, midx, rows, sem):
        wid = lax.axis_index("s") * NC + lax.axis_index("c")
        base = wid * b_per_w
        pltpu.sync_copy(uid.at[pl.ds(base, b_per_w)], uidx)
        pltpu.sync_copy(mid.at[pl.ds(base, b_per_w)], midx)
        pltpu.async_copy(utab.at[uidx], rows, sem).wait()
        pltpu.sync_copy(rows, gu_out.at[pl.ds(base, b_per_w)])
        pltpu.async_copy(mtab.at[midx], rows, sem).wait()
        pltpu.sync_copy(rows, gm_out.at[pl.ds(base, b_per_w)])

    return gather_k


def _mlp_body(gu, gm, qu, qm, s1u, s1m, b1, w2, b2, wout, bout, out):
    lane_q = lax.broadcasted_iota(jnp.int32, (BB, 128), 1) // D
    xu = jnp.where(lane_q == qu[...], gu[...], 0.0)
    xm = jnp.where(lane_q == qm[...], gm[...], 0.0)
    h1 = (jnp.dot(xu, s1u[...], preferred_element_type=jnp.float32)
          + jnp.dot(xm, s1m[...], preferred_element_type=jnp.float32))
    h1 = jnp.maximum(h1 + b1[...], 0.0)
    dn = (((1,), (1,)), ((), ()))
    h2 = jnp.maximum(
        lax.dot_general(h1, w2[...], dn, preferred_element_type=jnp.float32)
        + b2[...], 0.0)
    out[...] = lax.dot_general(
        h2, wout[...], dn, preferred_element_type=jnp.float32) + bout[...]


def kernel(User_ID, Movie_ID, Rating, user_table, movie_table,
           W1, b1, W2, b2, Wout, bout):
    nu, nm = user_table.shape[0], movie_table.shape[0]
    ut4 = user_table.reshape(nu // PK, 128)
    mt4 = movie_table.reshape(nm // PK, 128)
    uid = User_ID.astype(jnp.int32)
    mid = Movie_ID.astype(jnp.int32)
    gu, gm = _gather_fn(nu // PK, nm // PK)(ut4, mt4, uid // PK, mid // PK)

    s1u = jnp.tile(W1[:, :D].T, (PK, 1))
    s1m = jnp.tile(W1[:, D:].T, (PK, 1))
    qu = (uid % PK).reshape(B, 1)
    qm = (mid % PK).reshape(B, 1)

    out = pl.pallas_call(
        _mlp_body,
        grid=(B // BB,),
        in_specs=[
            pl.BlockSpec((BB, 128), lambda i: (i, 0)),
            pl.BlockSpec((BB, 128), lambda i: (i, 0)),
            pl.BlockSpec((BB, 1), lambda i: (i, 0)),
            pl.BlockSpec((BB, 1), lambda i: (i, 0)),
            pl.BlockSpec((128, H), lambda i: (0, 0)),
            pl.BlockSpec((128, H), lambda i: (0, 0)),
            pl.BlockSpec((1, H), lambda i: (0, 0)),
            pl.BlockSpec((H, H), lambda i: (0, 0)),
            pl.BlockSpec((1, H), lambda i: (0, 0)),
            pl.BlockSpec((O, H), lambda i: (0, 0)),
            pl.BlockSpec((1, O), lambda i: (0, 0)),
        ],
        out_specs=pl.BlockSpec((BB, O), lambda i: (i, 0)),
        out_shape=jax.ShapeDtypeStruct((B, O), jnp.float32),
    )(gu, gm, qu, qm, s1u, s1m, b1.reshape(1, H), W2, b2.reshape(1, H),
      Wout, bout.reshape(1, O))
    return out


# speedup vs baseline: 1.0042x; 1.0042x over previous
import functools

import jax
import jax.numpy as jnp
from jax import lax
from jax.experimental import pallas as pl
from jax.experimental.pallas import tpu as pltpu
from jax.experimental.pallas import tpu_sc as plsc

B = 16384
D = 32
H = 128
O = 5
BB = 2048


@functools.cache
def _gather_fn(b):
    info = plsc.get_sparse_core_info()
    NC, NS = info.num_cores, info.num_subcores
    NW = NC * NS
    b_per_w = b // NW
    mesh = plsc.VectorSubcoreMesh(core_axis_name="c", subcore_axis_name="s")

    @functools.partial(
        pl.kernel,
        mesh=mesh,
        out_type=(
            jax.ShapeDtypeStruct((b, D), jnp.float32),
            jax.ShapeDtypeStruct((b, D), jnp.float32),
        ),
        scratch_types=[
            pltpu.VMEM((b_per_w,), jnp.int32),
            pltpu.VMEM((b_per_w,), jnp.int32),
            pltpu.VMEM((b_per_w, D), jnp.float32),
            pltpu.VMEM((b_per_w, D), jnp.float32),
            pltpu.SemaphoreType.DMA,
        ],
        compiler_params=pltpu.CompilerParams(use_tc_tiling_on_sc=False),
    )
    def gather_k(utab, mtab, uid, mid, ue_out, me_out,
                 uidx, midx, urows, mrows, sem):
        wid = lax.axis_index("s") * NC + lax.axis_index("c")
        base = wid * b_per_w
        pltpu.sync_copy(uid.at[pl.ds(base, b_per_w)], uidx)
        pltpu.sync_copy(mid.at[pl.ds(base, b_per_w)], midx)
        cu = pltpu.async_copy(utab.at[uidx], urows, sem)
        cm = pltpu.async_copy(mtab.at[midx], mrows, sem)
        cu.wait()
        cm.wait()
        pltpu.sync_copy(urows, ue_out.at[pl.ds(base, b_per_w)])
        pltpu.sync_copy(mrows, me_out.at[pl.ds(base, b_per_w)])

    return gather_k


def _mlp_body(ue, me, w1, b1, w2, b2, wout, bout, out):
    dn = (((1,), (1,)), ((), ()))
    h1 = lax.dot_general(ue[...], w1[:, :D], dn,
                         preferred_element_type=jnp.float32)
    h1 = h1 + lax.dot_general(me[...], w1[:, D:], dn,
                              preferred_element_type=jnp.float32)
    h1 = jnp.maximum(h1 + b1[...], 0.0)
    h2 = jnp.maximum(
        lax.dot_general(h1, w2[...], dn, preferred_element_type=jnp.float32)
        + b2[...], 0.0)
    out[...] = lax.dot_general(
        h2, wout[...], dn, preferred_element_type=jnp.float32) + bout[...]


def _mlp_call(ue, me, W1, b1, W2, b2, Wout, bout, b):
    return pl.pallas_call(
        _mlp_body,
        grid=(b // BB,),
        in_specs=[
            pl.BlockSpec((BB, D), lambda i: (i, 0)),
            pl.BlockSpec((BB, D), lambda i: (i, 0)),
            pl.BlockSpec((H, 2 * D), lambda i: (0, 0)),
            pl.BlockSpec((1, H), lambda i: (0, 0)),
            pl.BlockSpec((H, H), lambda i: (0, 0)),
            pl.BlockSpec((1, H), lambda i: (0, 0)),
            pl.BlockSpec((O, H), lambda i: (0, 0)),
            pl.BlockSpec((1, O), lambda i: (0, 0)),
        ],
        out_specs=pl.BlockSpec((BB, O), lambda i: (i, 0)),
        out_shape=jax.ShapeDtypeStruct((b, O), jnp.float32),
    )(ue, me, W1, b1.reshape(1, H), W2, b2.reshape(1, H),
      Wout, bout.reshape(1, O))


def kernel(User_ID, Movie_ID, Rating, user_table, movie_table,
           W1, b1, W2, b2, Wout, bout):
    uid = User_ID.astype(jnp.int32)
    mid = Movie_ID.astype(jnp.int32)
    hb = B // 2
    g = _gather_fn(hb)
    ue0, me0 = g(user_table, movie_table, uid[:hb], mid[:hb])
    ue1, me1 = g(user_table, movie_table, uid[hb:], mid[hb:])
    out0 = _mlp_call(ue0, me0, W1, b1, W2, b2, Wout, bout, hb)
    out1 = _mlp_call(ue1, me1, W1, b1, W2, b2, Wout, bout, hb)
    return jnp.concatenate([out0, out1], axis=0)


# speedup vs baseline: 1.0137x; 1.0095x over previous
import functools

import jax
import jax.numpy as jnp
from jax import lax
from jax.experimental import pallas as pl
from jax.experimental.pallas import tpu as pltpu
from jax.experimental.pallas import tpu_sc as plsc

B = 16384
D = 32
H = 128
O = 5
BB = 4096


@functools.cache
def _gather_fn():
    info = plsc.get_sparse_core_info()
    NC, NS = info.num_cores, info.num_subcores
    NW = NC * NS
    b_per_w = B // NW
    mesh = plsc.VectorSubcoreMesh(core_axis_name="c", subcore_axis_name="s")

    @functools.partial(
        pl.kernel,
        mesh=mesh,
        out_type=(
            jax.ShapeDtypeStruct((B, D), jnp.float32),
            jax.ShapeDtypeStruct((B, D), jnp.float32),
        ),
        scratch_types=[
            pltpu.VMEM((b_per_w,), jnp.int32),
            pltpu.VMEM((b_per_w,), jnp.int32),
            pltpu.VMEM((b_per_w, D), jnp.float32),
            pltpu.VMEM((b_per_w, D), jnp.float32),
            pltpu.SemaphoreType.DMA,
        ],
        compiler_params=pltpu.CompilerParams(use_tc_tiling_on_sc=False),
    )
    def gather_k(utab, mtab, uid, mid, ue_out, me_out,
                 uidx, midx, urows, mrows, sem):
        wid = lax.axis_index("s") * NC + lax.axis_index("c")
        base = wid * b_per_w
        pltpu.sync_copy(uid.at[pl.ds(base, b_per_w)], uidx)
        pltpu.sync_copy(mid.at[pl.ds(base, b_per_w)], midx)
        cu = pltpu.async_copy(utab.at[uidx], urows, sem)
        cm = pltpu.async_copy(mtab.at[midx], mrows, sem)
        cu.wait()
        cm.wait()
        pltpu.sync_copy(urows, ue_out.at[pl.ds(base, b_per_w)])
        pltpu.sync_copy(mrows, me_out.at[pl.ds(base, b_per_w)])

    return gather_k


def _mlp_body(ue, me, w1, b1, w2, b2, wout, bout, out):
    dn = (((1,), (1,)), ((), ()))
    h1 = lax.dot_general(ue[...], w1[:, :D], dn,
                         preferred_element_type=jnp.float32)
    h1 = h1 + lax.dot_general(me[...], w1[:, D:], dn,
                              preferred_element_type=jnp.float32)
    h1 = jnp.maximum(h1 + b1[...], 0.0)
    h2 = jnp.maximum(
        lax.dot_general(h1, w2[...], dn, preferred_element_type=jnp.float32)
        + b2[...], 0.0)
    out[...] = lax.dot_general(
        h2, wout[...], dn, preferred_element_type=jnp.float32) + bout[...]


def kernel(User_ID, Movie_ID, Rating, user_table, movie_table,
           W1, b1, W2, b2, Wout, bout):
    uid = User_ID.astype(jnp.int32)
    mid = Movie_ID.astype(jnp.int32)
    ue, me = _gather_fn()(user_table, movie_table, uid, mid)

    out = pl.pallas_call(
        _mlp_body,
        grid=(B // BB,),
        in_specs=[
            pl.BlockSpec((BB, D), lambda i: (i, 0)),
            pl.BlockSpec((BB, D), lambda i: (i, 0)),
            pl.BlockSpec((H, 2 * D), lambda i: (0, 0)),
            pl.BlockSpec((1, H), lambda i: (0, 0)),
            pl.BlockSpec((H, H), lambda i: (0, 0)),
            pl.BlockSpec((1, H), lambda i: (0, 0)),
            pl.BlockSpec((O, H), lambda i: (0, 0)),
            pl.BlockSpec((1, O), lambda i: (0, 0)),
        ],
        out_specs=pl.BlockSpec((BB, O), lambda i: (i, 0)),
        out_shape=jax.ShapeDtypeStruct((B, O), jnp.float32),
    )(ue, me, W1, b1.reshape(1, H), W2, b2.reshape(1, H),
      Wout, bout.reshape(1, O))
    return out
